# baseline (speedup 1.0000x reference)
import jax
import jax.numpy as jnp
from jax import lax
from jax.experimental import pallas as pl
from jax.experimental.pallas import tpu as pltpu
from jax.experimental.pallas import tpu_sc as plsc

KTOP = 8
B, S, C = 4, 4096, 1024
LANES = 16
BLK = 16
NBLK = S // BLK
CGROUPS = C // LANES
NUNITS = B * CGROUPS
NWORKERS = 32
UNITS_PER_W = NUNITS // NWORKERS


def _kmax_body(x_hbm, out_hbm, data_v, outb_v):
  wid = lax.axis_index("s") * 2 + lax.axis_index("c")
  iota = lax.iota(jnp.int32, LANES)
  neg = jnp.full((LANES,), -jnp.inf, jnp.float32)
  zeros_i = jnp.zeros((LANES,), jnp.int32)

  def unit_body(gi, _):
    g = wid * UNITS_PER_W + gi
    b = g // CGROUPS
    cg = g % CGROUPS

    pltpu.sync_copy(x_hbm.at[b, :, pl.ds(cg * LANES, LANES)], data_v)

    NACC = 4
    QB = NBLK // NACC

    def insert(ks, idxs, v, vi):
      for j in range(KTOP):
        m = v > ks[j]
        nk = jnp.where(m, v, ks[j])
        ni = jnp.where(m, vi, idxs[j])
        v = jnp.where(m, ks[j], v)
        vi = jnp.where(m, idxs[j], vi)
        ks[j] = nk
        idxs[j] = ni

    def blockmax(r0):
      vs = [data_v[r0 + i, :] for i in range(BLK)]
      while len(vs) > 1:
        vs = [jnp.maximum(vs[2 * t], vs[2 * t + 1]) for t in range(len(vs) // 2)]
      return vs[0]

    def p12(t, carry):
      sets = [[list(carry[q][0]), list(carry[q][1])] for q in range(NACC)]
      for q in range(NACC):
        k = QB * q + t
        v = blockmax(k * BLK)
        vi = jnp.broadcast_to(k, (LANES,)).astype(jnp.int32)
        insert(sets[q][0], sets[q][1], v, vi)
      return tuple((tuple(s[0]), tuple(s[1])) for s in sets)

    carry0 = tuple(((neg,) * KTOP, (zeros_i,) * KTOP) for _ in range(NACC))
    carry = lax.fori_loop(0, NBLK // NACC, p12, carry0)

    ks = list(carry[0][0])
    idxs = list(carry[0][1])
    for q in range(1, NACC):
      for j in range(KTOP):
        insert(ks, idxs, carry[q][0][j], carry[q][1][j])
    winners = idxs

    def p3(i, accs):
      a = list(accs[0])
      c = list(accs[1])
      for p in range(KTOP):
        row = winners[p] * BLK + i
        y = plsc.load_gather(data_v, [row, iota])
        tgt = a if p < KTOP // 2 else c
        for j in range(KTOP):
          hi = jnp.maximum(tgt[j], y)
          y = jnp.minimum(tgt[j], y)
          tgt[j] = hi
      return (tuple(a), tuple(c))

    acc2 = lax.fori_loop(0, BLK, p3, ((neg,) * KTOP, (neg,) * KTOP))
    accs = list(acc2[0])
    for p in range(KTOP):
      y = acc2[1][p]
      for j in range(KTOP):
        hi = jnp.maximum(accs[j], y)
        y = jnp.minimum(accs[j], y)
        accs[j] = hi

    for j in range(KTOP):
      plsc.store_scatter(outb_v, [iota * KTOP + j], accs[j])
    pltpu.sync_copy(outb_v, out_hbm.at[b, pl.ds(cg * LANES * KTOP, LANES * KTOP)])
    return 0

  lax.fori_loop(0, UNITS_PER_W, unit_body, 0)


def kernel(inputs):
  mesh = plsc.VectorSubcoreMesh(core_axis_name="c", subcore_axis_name="s")
  return pl.kernel(
      _kmax_body,
      out_type=jax.ShapeDtypeStruct((B, C * KTOP), jnp.float32),
      mesh=mesh,
      compiler_params=pltpu.CompilerParams(
          use_tc_tiling_on_sc=False, needs_layout_passes=False),
      scratch_types=[
          pltpu.VMEM((S, LANES), jnp.float32),
          pltpu.VMEM((LANES * KTOP,), jnp.float32),
      ],
  )(inputs)


# speedup vs baseline: 1.0616x; 1.0616x over previous
import jax
import jax.numpy as jnp
from jax import lax
from jax.experimental import pallas as pl
from jax.experimental.pallas import tpu as pltpu
from jax.experimental.pallas import tpu_sc as plsc

KTOP = 8
B, S, C = 4, 4096, 1024
LANES = 16
BLK = 16
NBLK = S // BLK
CGROUPS = C // LANES
NUNITS = B * CGROUPS
NWORKERS = 32
UNITS_PER_W = NUNITS // NWORKERS


def _kmax_body(x_hbm, out_hbm, data_v, outb_v):
  wid = lax.axis_index("s") * 2 + lax.axis_index("c")
  iota = lax.iota(jnp.int32, LANES)
  neg = jnp.full((LANES,), -jnp.inf, jnp.float32)
  zeros_i = jnp.zeros((LANES,), jnp.int32)

  def unit_body(gi, _):
    g = wid * UNITS_PER_W + gi
    b = g // CGROUPS
    cg = g % CGROUPS

    pltpu.sync_copy(x_hbm.at[b, :, pl.ds(cg * LANES, LANES)], data_v)

    NACC = 2
    QB = NBLK // NACC

    def insert(ks, idxs, v, vi):
      for j in range(KTOP):
        m = v > ks[j]
        nk = jnp.where(m, v, ks[j])
        ni = jnp.where(m, vi, idxs[j])
        v = jnp.where(m, ks[j], v)
        vi = jnp.where(m, idxs[j], vi)
        ks[j] = nk
        idxs[j] = ni

    def blockmax(r0):
      vs = [data_v[r0 + i, :] for i in range(BLK)]
      while len(vs) > 1:
        vs = [jnp.maximum(vs[2 * t], vs[2 * t + 1]) for t in range(len(vs) // 2)]
      return vs[0]

    def p12(t, carry):
      sets = [[list(carry[q][0]), list(carry[q][1])] for q in range(NACC)]
      for q in range(NACC):
        k = QB * q + t
        v = blockmax(k * BLK)
        vi = jnp.broadcast_to(k, (LANES,)).astype(jnp.int32)
        insert(sets[q][0], sets[q][1], v, vi)
      return tuple((tuple(s[0]), tuple(s[1])) for s in sets)

    carry0 = tuple(((neg,) * KTOP, (zeros_i,) * KTOP) for _ in range(NACC))
    carry = lax.fori_loop(0, NBLK // NACC, p12, carry0)

    ks = list(carry[0][0])
    idxs = list(carry[0][1])
    for q in range(1, NACC):
      for j in range(KTOP):
        insert(ks, idxs, carry[q][0][j], carry[q][1][j])
    winners = idxs

    def p3(i, accs):
      a = list(accs[0])
      c = list(accs[1])
      for p in range(KTOP):
        row = winners[p] * BLK + i
        y = plsc.load_gather(data_v, [row, iota])
        tgt = a if p < KTOP // 2 else c
        for j in range(KTOP):
          hi = jnp.maximum(tgt[j], y)
          y = jnp.minimum(tgt[j], y)
          tgt[j] = hi
      return (tuple(a), tuple(c))

    acc2 = lax.fori_loop(0, BLK, p3, ((neg,) * KTOP, (neg,) * KTOP))
    accs = list(acc2[0])
    for p in range(KTOP):
      y = acc2[1][p]
      for j in range(KTOP):
        hi = jnp.maximum(accs[j], y)
        y = jnp.minimum(accs[j], y)
        accs[j] = hi

    for j in range(KTOP):
      plsc.store_scatter(outb_v, [iota * KTOP + j], accs[j])
    pltpu.sync_copy(outb_v, out_hbm.at[b, pl.ds(cg * LANES * KTOP, LANES * KTOP)])
    return 0

  lax.fori_loop(0, UNITS_PER_W, unit_body, 0)


def kernel(inputs):
  mesh = plsc.VectorSubcoreMesh(core_axis_name="c", subcore_axis_name="s")
  return pl.kernel(
      _kmax_body,
      out_type=jax.ShapeDtypeStruct((B, C * KTOP), jnp.float32),
      mesh=mesh,
      compiler_params=pltpu.CompilerParams(
          use_tc_tiling_on_sc=False, needs_layout_passes=False),
      scratch_types=[
          pltpu.VMEM((S, LANES), jnp.float32),
          pltpu.VMEM((LANES * KTOP,), jnp.float32),
      ],
  )(inputs)


# speedup vs baseline: 1.1121x; 1.0476x over previous
import jax
import jax.numpy as jnp
from jax import lax
from jax.experimental import pallas as pl
from jax.experimental.pallas import tpu as pltpu
from jax.experimental.pallas import tpu_sc as plsc

KTOP = 8
B, S, C = 4, 4096, 1024
LANES = 16
BLK = 16
CHUNK = 256
NCHUNK = S // CHUNK
CBLK = C // 128
NGRP = 128 // LANES
BPC = CHUNK // BLK


def _kmax_body(x_hbm, out_hbm, data_v, res_v, outb_v):
  wid = lax.axis_index("s") * 2 + lax.axis_index("c")
  b = wid // CBLK
  cb = wid % CBLK
  iota = lax.iota(jnp.int32, LANES)
  neg = jnp.full((LANES,), -jnp.inf, jnp.float32)
  zeros_i = jnp.zeros((LANES,), jnp.int32)

  def insert_kv(ks, idxs, v, vi):
    for j in range(KTOP):
      m = v > ks[j]
      nk = jnp.where(m, v, ks[j])
      ni = jnp.where(m, vi, idxs[j])
      v = jnp.where(m, ks[j], v)
      vi = jnp.where(m, idxs[j], vi)
      ks[j] = nk
      idxs[j] = ni

  def chunk_body(ch, _):
    pltpu.sync_copy(
        x_hbm.at[b, pl.ds(pl.multiple_of(ch * CHUNK, CHUNK), CHUNK),
                 pl.ds(cb * 128, 128)],
        data_v)

    def group_body(g, _):
      col0 = pl.multiple_of(g * LANES, LANES)
      cols = col0 + iota

      def p12(k, carry):
        ks = list(carry[:KTOP])
        idxs = list(carry[KTOP:])
        r0 = k * BLK
        vs = [data_v[r0 + i, pl.ds(col0, LANES)] for i in range(BLK)]
        while len(vs) > 1:
          vs = [jnp.maximum(vs[2 * t], vs[2 * t + 1])
                for t in range(len(vs) // 2)]
        vi = jnp.broadcast_to(k, (LANES,)).astype(jnp.int32)
        insert_kv(ks, idxs, vs[0], vi)
        return tuple(ks) + tuple(idxs)

      carry0 = (neg,) * KTOP + (zeros_i,) * KTOP
      carry = lax.fori_loop(0, BPC, p12, carry0, unroll=2)
      winners = carry[KTOP:]

      def p3(i, accs):
        a = list(accs[0])
        c = list(accs[1])
        for p in range(KTOP):
          row = winners[p] * BLK + i
          y = plsc.load_gather(data_v, [row, cols])
          tgt = a if p < KTOP // 2 else c
          for j in range(KTOP):
            hi = jnp.maximum(tgt[j], y)
            y = jnp.minimum(tgt[j], y)
            tgt[j] = hi
        return (tuple(a), tuple(c))

      acc2 = lax.fori_loop(0, BLK, p3, ((neg,) * KTOP, (neg,) * KTOP))
      accs = list(acc2[0])
      for p in range(KTOP):
        y = acc2[1][p]
        for j in range(KTOP):
          hi = jnp.maximum(accs[j], y)
          y = jnp.minimum(accs[j], y)
          accs[j] = hi

      for j in range(KTOP):
        res_v[ch * KTOP + j, pl.ds(col0, LANES)] = accs[j]
      return 0

    lax.fori_loop(0, NGRP, group_body, 0)
    return 0

  lax.fori_loop(0, NCHUNK, chunk_body, 0)

  def final_body(g, _):
    col0 = pl.multiple_of(g * LANES, LANES)
    accs = [neg] * KTOP

    def merge(ch, carry):
      accs = list(carry)
      for j in range(KTOP):
        y = res_v[ch * KTOP + j, pl.ds(col0, LANES)]
        for t in range(KTOP):
          hi = jnp.maximum(accs[t], y)
          y = jnp.minimum(accs[t], y)
          accs[t] = hi
      return tuple(accs)

    accs = lax.fori_loop(0, NCHUNK, merge, tuple(accs))
    for j in range(KTOP):
      plsc.store_scatter(outb_v, [(col0 + iota) * KTOP + j], accs[j])
    return 0

  lax.fori_loop(0, NGRP, final_body, 0)
  pltpu.sync_copy(outb_v, out_hbm.at[b, pl.ds(cb * 1024, 1024)])


def kernel(inputs):
  mesh = plsc.VectorSubcoreMesh(core_axis_name="c", subcore_axis_name="s")
  return pl.kernel(
      _kmax_body,
      out_type=jax.ShapeDtypeStruct((B, C * KTOP), jnp.float32),
      mesh=mesh,
      compiler_params=pltpu.CompilerParams(needs_layout_passes=False),
      scratch_types=[
          pltpu.VMEM((CHUNK, 128), jnp.float32),
          pltpu.VMEM((NCHUNK * KTOP, 128), jnp.float32),
          pltpu.VMEM((1024,), jnp.float32),
      ],
  )(inputs)


# speedup vs baseline: 1.4901x; 1.3399x over previous
import jax
import jax.numpy as jnp
from jax import lax
from jax.experimental import pallas as pl
from jax.experimental.pallas import tpu as pltpu
from jax.experimental.pallas import tpu_sc as plsc

KTOP = 8
B, S, C = 4, 4096, 1024
LANES = 16
BLK = 16
CHUNK = 512
NCHUNK = S // CHUNK
CBLK = C // 128
NGRP = 128 // LANES
BPC = CHUNK // BLK


def _kmax_body(x_hbm, out_hbm, data_v, res_v, outb_v):
  wid = lax.axis_index("s") * 2 + lax.axis_index("c")
  b = wid // CBLK
  cb = wid % CBLK
  iota = lax.iota(jnp.int32, LANES)
  neg = jnp.full((LANES,), -jnp.inf, jnp.float32)
  zeros_i = jnp.zeros((LANES,), jnp.int32)

  def insert_kv(ks, idxs, v, vi):
    for j in range(KTOP):
      m = v > ks[j]
      nk = jnp.where(m, v, ks[j])
      ni = jnp.where(m, vi, idxs[j])
      v = jnp.where(m, ks[j], v)
      vi = jnp.where(m, idxs[j], vi)
      ks[j] = nk
      idxs[j] = ni

  def chunk_body(ch, _):
    pltpu.sync_copy(
        x_hbm.at[b, pl.ds(pl.multiple_of(ch * CHUNK, CHUNK), CHUNK),
                 pl.ds(cb * 128, 128)],
        data_v)

    def group_body(g, _):
      col0 = pl.multiple_of(g * LANES, LANES)
      cols = col0 + iota

      def p12(k, carry):
        ks = list(carry[:KTOP])
        idxs = list(carry[KTOP:])
        r0 = k * BLK
        vs = [data_v[r0 + i, pl.ds(col0, LANES)] for i in range(BLK)]
        while len(vs) > 1:
          vs = [jnp.maximum(vs[2 * t], vs[2 * t + 1])
                for t in range(len(vs) // 2)]
        vi = jnp.broadcast_to(k, (LANES,)).astype(jnp.int32)
        insert_kv(ks, idxs, vs[0], vi)
        return tuple(ks) + tuple(idxs)

      carry0 = (neg,) * KTOP + (zeros_i,) * KTOP
      carry = lax.fori_loop(0, BPC, p12, carry0, unroll=2)
      winners = carry[KTOP:]

      def p3(i, accs):
        a = list(accs[0])
        c = list(accs[1])
        for p in range(KTOP):
          row = winners[p] * BLK + i
          y = plsc.load_gather(data_v, [row, cols])
          tgt = a if p < KTOP // 2 else c
          for j in range(KTOP):
            hi = jnp.maximum(tgt[j], y)
            y = jnp.minimum(tgt[j], y)
            tgt[j] = hi
        return (tuple(a), tuple(c))

      acc2 = lax.fori_loop(0, BLK, p3, ((neg,) * KTOP, (neg,) * KTOP))
      accs = list(acc2[0])
      for p in range(KTOP):
        y = acc2[1][p]
        for j in range(KTOP):
          hi = jnp.maximum(accs[j], y)
          y = jnp.minimum(accs[j], y)
          accs[j] = hi

      for j in range(KTOP):
        res_v[ch * KTOP + j, pl.ds(col0, LANES)] = accs[j]
      return 0

    lax.fori_loop(0, NGRP, group_body, 0)
    return 0

  lax.fori_loop(0, NCHUNK, chunk_body, 0)

  def final_body(g, _):
    col0 = pl.multiple_of(g * LANES, LANES)
    accs = [neg] * KTOP

    def merge(ch, carry):
      accs = list(carry)
      for j in range(KTOP):
        y = res_v[ch * KTOP + j, pl.ds(col0, LANES)]
        for t in range(KTOP):
          hi = jnp.maximum(accs[t], y)
          y = jnp.minimum(accs[t], y)
          accs[t] = hi
      return tuple(accs)

    accs = lax.fori_loop(0, NCHUNK, merge, tuple(accs))
    for j in range(KTOP):
      plsc.store_scatter(outb_v, [(col0 + iota) * KTOP + j], accs[j])
    return 0

  lax.fori_loop(0, NGRP, final_body, 0)
  pltpu.sync_copy(outb_v, out_hbm.at[b, pl.ds(cb * 1024, 1024)])


def kernel(inputs):
  mesh = plsc.VectorSubcoreMesh(core_axis_name="c", subcore_axis_name="s")
  return pl.kernel(
      _kmax_body,
      out_type=jax.ShapeDtypeStruct((B, C * KTOP), jnp.float32),
      mesh=mesh,
      compiler_params=pltpu.CompilerParams(needs_layout_passes=False),
      scratch_types=[
          pltpu.VMEM((CHUNK, 128), jnp.float32),
          pltpu.VMEM((NCHUNK * KTOP, 128), jnp.float32),
          pltpu.VMEM((1024,), jnp.float32),
      ],
  )(inputs)


# speedup vs baseline: 1.7536x; 1.1768x over previous
import jax
import jax.numpy as jnp
from jax import lax
from jax.experimental import pallas as pl
from jax.experimental.pallas import tpu as pltpu
from jax.experimental.pallas import tpu_sc as plsc

KTOP = 8
B, S, C = 4, 4096, 1024
LANES = 16
BLK = 8
CHUNK = 512
NCHUNK = S // CHUNK
CBLK = C // 128
NGRP = 128 // LANES
BPC = CHUNK // BLK


def _kmax_body(x_hbm, out_hbm, data_v, res_v, outb_v):
  wid = lax.axis_index("s") * 2 + lax.axis_index("c")
  b = wid // CBLK
  cb = wid % CBLK
  iota = lax.iota(jnp.int32, LANES)
  neg = jnp.full((LANES,), -jnp.inf, jnp.float32)
  zkey = jnp.zeros((LANES,), jnp.uint32)

  def chunk_body(ch, _):
    pltpu.sync_copy(
        x_hbm.at[b, pl.ds(pl.multiple_of(ch * CHUNK, CHUNK), CHUNK),
                 pl.ds(cb * 128, 128)],
        data_v)

    def group_body(g, _):
      col0 = pl.multiple_of(g * LANES, LANES)
      cols = col0 + iota

      def p12(k, ks):
        ks = list(ks)
        r0 = k * BLK
        vs = [data_v[r0 + i, pl.ds(col0, LANES)] for i in range(BLK)]
        while len(vs) > 1:
          vs = [jnp.maximum(vs[2 * t], vs[2 * t + 1])
                for t in range(len(vs) // 2)]
        bits = lax.bitcast_convert_type(vs[0], jnp.uint32)
        sgn = lax.bitcast_convert_type(
            lax.shift_right_arithmetic(
                lax.bitcast_convert_type(bits, jnp.int32), 31), jnp.uint32)
        u = bits ^ (sgn | jnp.uint32(0x80000000))
        key = (u & jnp.uint32(0xFFFFFFC0)) | jnp.broadcast_to(
            k, (LANES,)).astype(jnp.uint32)
        for j in range(KTOP):
          hi = jnp.maximum(ks[j], key)
          key = jnp.minimum(ks[j], key)
          ks[j] = hi
        return tuple(ks)

      ks = lax.fori_loop(0, BPC, p12, (zkey,) * KTOP, unroll=2)

      rowbase = [
          lax.bitcast_convert_type(ks[p] & jnp.uint32(63), jnp.int32) * BLK
          for p in range(KTOP)
      ]

      def p3(i, accs):
        a = list(accs[0])
        c = list(accs[1])
        for p in range(KTOP):
          y = plsc.load_gather(data_v, [rowbase[p] + i, cols])
          tgt = a if p < KTOP // 2 else c
          for j in range(KTOP):
            hi = jnp.maximum(tgt[j], y)
            y = jnp.minimum(tgt[j], y)
            tgt[j] = hi
        return (tuple(a), tuple(c))

      acc2 = lax.fori_loop(0, BLK, p3, ((neg,) * KTOP, (neg,) * KTOP))
      accs = list(acc2[0])
      for p in range(KTOP):
        y = acc2[1][p]
        for j in range(KTOP):
          hi = jnp.maximum(accs[j], y)
          y = jnp.minimum(accs[j], y)
          accs[j] = hi

      for j in range(KTOP):
        res_v[ch * KTOP + j, pl.ds(col0, LANES)] = accs[j]
      return 0

    lax.fori_loop(0, NGRP, group_body, 0)
    return 0

  lax.fori_loop(0, NCHUNK, chunk_body, 0)

  def final_body(g, _):
    col0 = pl.multiple_of(g * LANES, LANES)

    def merge(ch, carry):
      accs = list(carry)
      for j in range(KTOP):
        y = res_v[ch * KTOP + j, pl.ds(col0, LANES)]
        for t in range(KTOP):
          hi = jnp.maximum(accs[t], y)
          y = jnp.minimum(accs[t], y)
          accs[t] = hi
      return tuple(accs)

    accs = lax.fori_loop(0, NCHUNK, merge, (neg,) * KTOP)
    for j in range(KTOP):
      plsc.store_scatter(outb_v, [(col0 + iota) * KTOP + j], accs[j])
    return 0

  lax.fori_loop(0, NGRP, final_body, 0)
  pltpu.sync_copy(outb_v, out_hbm.at[b, pl.ds(cb * 1024, 1024)])


def kernel(inputs):
  mesh = plsc.VectorSubcoreMesh(core_axis_name="c", subcore_axis_name="s")
  return pl.kernel(
      _kmax_body,
      out_type=jax.ShapeDtypeStruct((B, C * KTOP), jnp.float32),
      mesh=mesh,
      compiler_params=pltpu.CompilerParams(needs_layout_passes=False),
      scratch_types=[
          pltpu.VMEM((CHUNK, 128), jnp.float32),
          pltpu.VMEM((NCHUNK * KTOP, 128), jnp.float32),
          pltpu.VMEM((1024,), jnp.float32),
      ],
  )(inputs)


# speedup vs baseline: 1.7877x; 1.0194x over previous
import jax
import jax.numpy as jnp
from jax import lax
from jax.experimental import pallas as pl
from jax.experimental.pallas import tpu as pltpu
from jax.experimental.pallas import tpu_sc as plsc

KTOP = 8
B, S, C = 4, 4096, 1024
LANES = 16
BLK = 8
CHUNK = 512
NCHUNK = S // CHUNK
CBLK = C // 128
NGRP = 128 // LANES
BPC = CHUNK // BLK


def _kmax_body(x_hbm, out_hbm, data_v, res_v, keys_v, outb_v, sem_a, sem_b):
  wid = lax.axis_index("s") * 2 + lax.axis_index("c")
  b = wid // CBLK
  cb = wid % CBLK
  iota = lax.iota(jnp.int32, LANES)
  neg = jnp.full((LANES,), -jnp.inf, jnp.float32)
  zkey = jnp.zeros((LANES,), jnp.uint32)

  NQ = 4
  QROWS = CHUNK // NQ
  QBLK = QROWS // BLK

  def chunk_body(ch, _):
    r0hbm = pl.multiple_of(ch * CHUNK, CHUNK)

    def start_q(q, sem):
      return pltpu.async_copy(
          x_hbm.at[b, pl.ds(pl.multiple_of(r0hbm + q * QROWS, QROWS), QROWS),
                   pl.ds(cb * 128, 128)],
          data_v.at[pl.ds(q * QROWS, QROWS), :],
          sem)

    hs = {0: start_q(0, sem_a)}
    for q in range(NQ):
      hs[q].wait()
      if q + 1 < NQ:
        hs[q + 1] = start_q(q + 1, sem_a if (q + 1) % 2 == 0 else sem_b)

      def q_body(g, _, q=q):
        col0 = pl.multiple_of(g * LANES, LANES)
        if q == 0:
          ks = [zkey] * KTOP
        else:
          ks = [
              lax.bitcast_convert_type(
                  keys_v[g * KTOP + j, pl.ds(0, LANES)], jnp.uint32)
              for j in range(KTOP)
          ]
        for k in range(QBLK):
          r0 = q * QROWS + k * BLK
          vs = [data_v[r0 + i, pl.ds(col0, LANES)] for i in range(BLK)]
          while len(vs) > 1:
            vs = [jnp.maximum(vs[2 * t], vs[2 * t + 1])
                  for t in range(len(vs) // 2)]
          bits = lax.bitcast_convert_type(vs[0], jnp.uint32)
          sgn = lax.bitcast_convert_type(
              lax.shift_right_arithmetic(
                  lax.bitcast_convert_type(bits, jnp.int32), 31), jnp.uint32)
          u = bits ^ (sgn | jnp.uint32(0x80000000))
          key = (u & jnp.uint32(0xFFFFFFC0)) | jnp.uint32(q * QBLK + k)
          for j in range(KTOP):
            hi = jnp.maximum(ks[j], key)
            key = jnp.minimum(ks[j], key)
            ks[j] = hi
        for j in range(KTOP):
          keys_v[g * KTOP + j, pl.ds(0, LANES)] = lax.bitcast_convert_type(
              ks[j], jnp.float32)
        return 0

      lax.fori_loop(0, NGRP, q_body, 0)

    def p3_body(g, _):
      col0 = pl.multiple_of(g * LANES, LANES)
      cols = col0 + iota
      ks = [
          lax.bitcast_convert_type(
              keys_v[g * KTOP + j, pl.ds(0, LANES)], jnp.uint32)
          for j in range(KTOP)
      ]
      rowbase = [
          lax.bitcast_convert_type(ks[p] & jnp.uint32(63), jnp.int32) * BLK
          for p in range(KTOP)
      ]
      a = [neg] * KTOP
      c = [neg] * KTOP
      for i in range(BLK):
        for p in range(KTOP):
          y = plsc.load_gather(data_v, [rowbase[p] + i, cols])
          tgt = a if p < KTOP // 2 else c
          for j in range(KTOP):
            hi = jnp.maximum(tgt[j], y)
            y = jnp.minimum(tgt[j], y)
            tgt[j] = hi
      for p in range(KTOP):
        y = c[p]
        for j in range(KTOP):
          hi = jnp.maximum(a[j], y)
          y = jnp.minimum(a[j], y)
          a[j] = hi
      for j in range(KTOP):
        res_v[ch * KTOP + j, pl.ds(col0, LANES)] = a[j]
      return 0

    lax.fori_loop(0, NGRP, p3_body, 0)
    return 0

  lax.fori_loop(0, NCHUNK, chunk_body, 0)

  def final_body(g, _):
    col0 = pl.multiple_of(g * LANES, LANES)

    def merge(ch, carry):
      accs = list(carry)
      for j in range(KTOP):
        y = res_v[ch * KTOP + j, pl.ds(col0, LANES)]
        for t in range(KTOP):
          hi = jnp.maximum(accs[t], y)
          y = jnp.minimum(accs[t], y)
          accs[t] = hi
      return tuple(accs)

    accs = lax.fori_loop(0, NCHUNK, merge, (neg,) * KTOP)
    for j in range(KTOP):
      plsc.store_scatter(outb_v, [(col0 + iota) * KTOP + j], accs[j])
    return 0

  lax.fori_loop(0, NGRP, final_body, 0)
  pltpu.sync_copy(outb_v, out_hbm.at[b, pl.ds(cb * 1024, 1024)])


def kernel(inputs):
  mesh = plsc.VectorSubcoreMesh(core_axis_name="c", subcore_axis_name="s")
  return pl.kernel(
      _kmax_body,
      out_type=jax.ShapeDtypeStruct((B, C * KTOP), jnp.float32),
      mesh=mesh,
      compiler_params=pltpu.CompilerParams(needs_layout_passes=False),
      scratch_types=[
          pltpu.VMEM((CHUNK, 128), jnp.float32),
          pltpu.VMEM((NCHUNK * KTOP, 128), jnp.float32),
          pltpu.VMEM((NGRP * KTOP, 128), jnp.float32),
          pltpu.VMEM((1024,), jnp.float32),
          pltpu.SemaphoreType.DMA,
          pltpu.SemaphoreType.DMA,
      ],
  )(inputs)
